# baseline (speedup 1.0000x reference)
import functools

import jax
import jax.numpy as jnp
from jax import lax
from jax.experimental import pallas as pl
from jax.experimental.pallas import tpu as pltpu
from jax.experimental.pallas import tpu_sc as plsc

B = 128


def _ln(v, eps=1e-5):
    mu = jnp.mean(v, axis=-1, keepdims=True)
    var = jnp.mean((v - mu) ** 2, axis=-1, keepdims=True)
    return (v - mu) * lax.rsqrt(var + eps)


def _changed(bc_ref, i):
    return (i == 0) | (bc_ref[i] != bc_ref[jnp.maximum(i - 1, 0)])


def _sc_gather_rows(table, idx):
    rows_out = idx.shape[0]
    d = table.shape[1]
    info = plsc.get_sparse_core_info()
    nw = info.num_cores * info.num_subcores
    rpw = rows_out // nw
    assert rows_out % nw == 0 and rpw % 8 == 0
    max_chunk = (448 * 1024) // (d * table.dtype.itemsize)
    chunk = max(k for k in range(8, max_chunk + 1, 8) if rpw % k == 0)
    nch = rpw // chunk
    mesh = plsc.VectorSubcoreMesh(core_axis_name="c", subcore_axis_name="s")

    @functools.partial(
        pl.kernel,
        mesh=mesh,
        out_type=jax.ShapeDtypeStruct((rows_out, d), table.dtype),
        scratch_types=[
            pltpu.VMEM((chunk,), jnp.int32),
            pltpu.VMEM((chunk, d), table.dtype),
            pltpu.SemaphoreType.DMA,
        ],
    )
    def k(table_hbm, idx_hbm, out_hbm, idx_v, rows_v, sem):
        wid = lax.axis_index("s") * info.num_cores + lax.axis_index("c")
        base = wid * rpw

        def body(i, carry):
            off = base + i * chunk
            pltpu.sync_copy(idx_hbm.at[pl.ds(off, chunk)], idx_v)
            pltpu.async_copy(table_hbm.at[idx_v], rows_v, sem).wait()
            pltpu.sync_copy(rows_v, out_hbm.at[pl.ds(off, chunk)])
            return carry

        lax.fori_loop(0, nch, body, 0)

    return k(table, idx)


def _k1_body(bc_ref, xs_ref, w1_ref, b1_ref, o_ref, wc_ref):
    @pl.when(_changed(bc_ref, pl.program_id(0)))
    def _():
        wc_ref[...] = w1_ref[0].astype(jnp.bfloat16)

    xn = _ln(xs_ref[...]).astype(jnp.bfloat16)
    h = jnp.dot(xn, wc_ref[...], preferred_element_type=jnp.float32) + b1_ref[0]
    o_ref[...] = jnp.maximum(h, 0.0).astype(jnp.bfloat16)


def _k2_body(bc_ref, h1_ref, wm_ref, wg_ref, bm_ref, bg_ref, o_ref,
             wmc_ref, wgc_ref):
    @pl.when(_changed(bc_ref, pl.program_id(1)))
    def _():
        wmc_ref[...] = wm_ref[0].astype(jnp.bfloat16)
        wgc_ref[...] = wg_ref[0].astype(jnp.bfloat16)

    h1 = h1_ref[...]
    main = jnp.dot(h1, wmc_ref[...], preferred_element_type=jnp.float32)
    gate = jnp.dot(h1, wgc_ref[...], preferred_element_type=jnp.float32)
    main = main + bm_ref[0]
    gate = gate + bg_ref[0]
    o_ref[...] = (main * jax.nn.sigmoid(gate)).astype(jnp.bfloat16)


def _k3_body(bc_ref, u_ref, wo_ref, bo_ref, o_ref, wc_ref):
    @pl.when(_changed(bc_ref, pl.program_id(0)))
    def _():
        wc_ref[...] = wo_ref[0].astype(jnp.bfloat16)

    g = _ln(u_ref[...].astype(jnp.float32)).astype(jnp.bfloat16)
    h = jnp.dot(g, wc_ref[...], preferred_element_type=jnp.float32) + bo_ref[0]
    o_ref[...] = h.astype(jnp.bfloat16)


def _k4_body(bc_ref, h2_ref, w2_ref, b2_ref, xs_ref, o_ref, wc_ref):
    @pl.when(_changed(bc_ref, pl.program_id(0)))
    def _():
        wc_ref[...] = w2_ref[0].astype(jnp.bfloat16)

    h = _ln(h2_ref[...].astype(jnp.float32)).astype(jnp.bfloat16)
    o = jnp.dot(h, wc_ref[...], preferred_element_type=jnp.float32) + b2_ref[0]
    o = o + 0.1 * xs_ref[...]
    o_ref[...] = _ln(o)


def kernel(x, cat_ids, W1, b1, Wm, bm, Wg, bg, Wo, bo, W2, b2):
    n, d = x.shape
    c, _, h = W1.shape
    b1, bm, bg, bo, b2 = (v[:, None, :] for v in (b1, bm, bg, bo, b2))
    n_pad = n + c * B
    nb = n_pad // B

    cat = cat_ids.astype(jnp.int32)
    onehot = (cat[:, None] == jnp.arange(c, dtype=jnp.int32)[None, :])
    ranks = jnp.cumsum(onehot.astype(jnp.int32), axis=0)
    counts = ranks[-1]
    rank = jnp.take_along_axis(ranks, cat[:, None], axis=1)[:, 0] - 1
    padded = ((counts + B - 1) // B) * B
    pad_start = jnp.concatenate(
        [jnp.zeros((1,), jnp.int32), jnp.cumsum(padded)[:-1].astype(jnp.int32)])
    slot = pad_start[cat] + rank
    src_idx = (jnp.arange(n_pad, dtype=jnp.int32) % n).at[slot].set(
        jnp.arange(n, dtype=jnp.int32))
    blocks_end = jnp.cumsum(padded // B).astype(jnp.int32)
    block_cat = jnp.minimum(
        jnp.searchsorted(blocks_end, jnp.arange(nb, dtype=jnp.int32),
                         side="right"),
        c - 1).astype(jnp.int32)

    xs = _sc_gather_rows(x, src_idx)

    h1 = pl.pallas_call(
        _k1_body,
        grid_spec=pltpu.PrefetchScalarGridSpec(
            num_scalar_prefetch=1,
            grid=(nb,),
            in_specs=[
                pl.BlockSpec((B, d), lambda i, bc: (i, 0)),
                pl.BlockSpec((1, d, h), lambda i, bc: (bc[i], 0, 0)),
                pl.BlockSpec((1, 1, h), lambda i, bc: (bc[i], 0, 0)),
            ],
            out_specs=pl.BlockSpec((B, h), lambda i, bc: (i, 0)),
            scratch_shapes=[pltpu.VMEM((d, h), jnp.bfloat16)],
        ),
        out_shape=jax.ShapeDtypeStruct((n_pad, h), jnp.bfloat16),
    )(block_cat, xs, W1, b1)

    th = h // 2
    u = pl.pallas_call(
        _k2_body,
        grid_spec=pltpu.PrefetchScalarGridSpec(
            num_scalar_prefetch=1,
            grid=(2, nb),
            in_specs=[
                pl.BlockSpec((B, h), lambda j, i, bc: (i, 0)),
                pl.BlockSpec((1, h, th), lambda j, i, bc: (bc[i], 0, j)),
                pl.BlockSpec((1, h, th), lambda j, i, bc: (bc[i], 0, j)),
                pl.BlockSpec((1, 1, th), lambda j, i, bc: (bc[i], 0, j)),
                pl.BlockSpec((1, 1, th), lambda j, i, bc: (bc[i], 0, j)),
            ],
            out_specs=pl.BlockSpec((B, th), lambda j, i, bc: (i, j)),
            scratch_shapes=[pltpu.VMEM((h, th), jnp.bfloat16),
                            pltpu.VMEM((h, th), jnp.bfloat16)],
        ),
        out_shape=jax.ShapeDtypeStruct((n_pad, h), jnp.bfloat16),
    )(block_cat, h1, Wm, Wg, bm, bg)

    h2 = pl.pallas_call(
        _k3_body,
        grid_spec=pltpu.PrefetchScalarGridSpec(
            num_scalar_prefetch=1,
            grid=(nb,),
            in_specs=[
                pl.BlockSpec((B, h), lambda i, bc: (i, 0)),
                pl.BlockSpec((1, h, h), lambda i, bc: (bc[i], 0, 0)),
                pl.BlockSpec((1, 1, h), lambda i, bc: (bc[i], 0, 0)),
            ],
            out_specs=pl.BlockSpec((B, h), lambda i, bc: (i, 0)),
            scratch_shapes=[pltpu.VMEM((h, h), jnp.bfloat16)],
        ),
        out_shape=jax.ShapeDtypeStruct((n_pad, h), jnp.bfloat16),
    )(block_cat, u, Wo, bo)

    ys = pl.pallas_call(
        _k4_body,
        grid_spec=pltpu.PrefetchScalarGridSpec(
            num_scalar_prefetch=1,
            grid=(nb,),
            in_specs=[
                pl.BlockSpec((B, h), lambda i, bc: (i, 0)),
                pl.BlockSpec((1, h, d), lambda i, bc: (bc[i], 0, 0)),
                pl.BlockSpec((1, 1, d), lambda i, bc: (bc[i], 0, 0)),
                pl.BlockSpec((B, d), lambda i, bc: (i, 0)),
            ],
            out_specs=pl.BlockSpec((B, d), lambda i, bc: (i, 0)),
            scratch_shapes=[pltpu.VMEM((h, d), jnp.bfloat16)],
        ),
        out_shape=jax.ShapeDtypeStruct((n_pad, d), jnp.float32),
    )(block_cat, h2, W2, b2, xs)

    return _sc_gather_rows(ys, slot)


# speedup vs baseline: 1.1593x; 1.1593x over previous
import functools

import jax
import jax.numpy as jnp
from jax import lax
from jax.experimental import pallas as pl
from jax.experimental.pallas import tpu as pltpu
from jax.experimental.pallas import tpu_sc as plsc

B = 256


def _ln(v, eps=1e-5):
    mu = jnp.mean(v, axis=-1, keepdims=True)
    var = jnp.mean((v - mu) ** 2, axis=-1, keepdims=True)
    return (v - mu) * lax.rsqrt(var + eps)


def _sc_gather_rows(table, idx):
    rows_out = idx.shape[0]
    d = table.shape[1]
    info = plsc.get_sparse_core_info()
    nw = info.num_cores * info.num_subcores
    rpw = rows_out // nw
    assert rows_out % nw == 0 and rpw % 8 == 0
    max_chunk = (448 * 1024) // (d * table.dtype.itemsize)
    chunk = max(k for k in range(8, max_chunk + 1, 8) if rpw % k == 0)
    nch = rpw // chunk
    mesh = plsc.VectorSubcoreMesh(core_axis_name="c", subcore_axis_name="s")

    @functools.partial(
        pl.kernel,
        mesh=mesh,
        out_type=jax.ShapeDtypeStruct((rows_out, d), table.dtype),
        scratch_types=[
            pltpu.VMEM((chunk,), jnp.int32),
            pltpu.VMEM((chunk, d), table.dtype),
            pltpu.SemaphoreType.DMA,
        ],
    )
    def k(table_hbm, idx_hbm, out_hbm, idx_v, rows_v, sem):
        wid = lax.axis_index("s") * info.num_cores + lax.axis_index("c")
        base = wid * rpw

        def body(i, carry):
            off = base + i * chunk
            pltpu.sync_copy(idx_hbm.at[pl.ds(off, chunk)], idx_v)
            pltpu.async_copy(table_hbm.at[idx_v], rows_v, sem).wait()
            pltpu.sync_copy(rows_v, out_hbm.at[pl.ds(off, chunk)])
            return carry

        lax.fori_loop(0, nch, body, 0)

    return k(table, idx)


def _k1_body(bc_ref, bv_ref, xs_ref, w1_ref, b1_ref, o_ref):
    @pl.when(bv_ref[pl.program_id(0)] > 0)
    def _():
        xn = _ln(xs_ref[...])
        h = jnp.dot(xn, w1_ref[0], preferred_element_type=jnp.float32)
        o_ref[...] = jnp.maximum(h + b1_ref[0], 0.0).astype(jnp.bfloat16)


def _k2_body(bc_ref, bv_ref, h1_ref, wm_ref, wg_ref, bm_ref, bg_ref, o_ref):
    @pl.when(bv_ref[pl.program_id(1)] > 0)
    def _():
        h1 = h1_ref[...].astype(jnp.float32)
        main = jnp.dot(h1, wm_ref[0], preferred_element_type=jnp.float32)
        gate = jnp.dot(h1, wg_ref[0], preferred_element_type=jnp.float32)
        main = main + bm_ref[0]
        gate = gate + bg_ref[0]
        o_ref[...] = (main * jax.nn.sigmoid(gate)).astype(jnp.bfloat16)


def _k3_body(bc_ref, bv_ref, u_ref, wo_ref, bo_ref, o_ref):
    @pl.when(bv_ref[pl.program_id(0)] > 0)
    def _():
        g = _ln(u_ref[...].astype(jnp.float32))
        h = jnp.dot(g, wo_ref[0], preferred_element_type=jnp.float32)
        o_ref[...] = (h + bo_ref[0]).astype(jnp.bfloat16)


def _k4_body(bc_ref, bv_ref, h2_ref, w2_ref, b2_ref, xs_ref, o_ref):
    @pl.when(bv_ref[pl.program_id(0)] > 0)
    def _():
        h = _ln(h2_ref[...].astype(jnp.float32))
        o = jnp.dot(h, w2_ref[0], preferred_element_type=jnp.float32) + b2_ref[0]
        o = o + 0.1 * xs_ref[...]
        o_ref[...] = _ln(o)


def kernel(x, cat_ids, W1, b1, Wm, bm, Wg, bg, Wo, bo, W2, b2):
    n, d = x.shape
    c, _, h = W1.shape
    b1, bm, bg, bo, b2 = (v[:, None, :] for v in (b1, bm, bg, bo, b2))
    n_pad = n + c * B
    nb = n_pad // B

    cat = cat_ids.astype(jnp.int32)
    onehot = (cat[:, None] == jnp.arange(c, dtype=jnp.int32)[None, :])
    ranks = jnp.cumsum(onehot.astype(jnp.int32), axis=0)
    counts = ranks[-1]
    rank = jnp.take_along_axis(ranks, cat[:, None], axis=1)[:, 0] - 1
    padded = ((counts + B - 1) // B) * B
    pad_start = jnp.concatenate(
        [jnp.zeros((1,), jnp.int32), jnp.cumsum(padded)[:-1].astype(jnp.int32)])
    slot = pad_start[cat] + rank
    src_idx = (jnp.arange(n_pad, dtype=jnp.int32) % n).at[slot].set(
        jnp.arange(n, dtype=jnp.int32))
    blocks_end = jnp.cumsum(padded // B).astype(jnp.int32)
    block_cat = jnp.minimum(
        jnp.searchsorted(blocks_end, jnp.arange(nb, dtype=jnp.int32),
                         side="right"),
        c - 1).astype(jnp.int32)
    block_valid = (jnp.arange(nb, dtype=jnp.int32)
                   < blocks_end[-1]).astype(jnp.int32)

    xs = _sc_gather_rows(x, src_idx)

    h1 = pl.pallas_call(
        _k1_body,
        grid_spec=pltpu.PrefetchScalarGridSpec(
            num_scalar_prefetch=2,
            grid=(nb,),
            in_specs=[
                pl.BlockSpec((B, d), lambda i, bc, bv: (i, 0)),
                pl.BlockSpec((1, d, h), lambda i, bc, bv: (bc[i], 0, 0)),
                pl.BlockSpec((1, 1, h), lambda i, bc, bv: (bc[i], 0, 0)),
            ],
            out_specs=pl.BlockSpec((B, h), lambda i, bc, bv: (i, 0)),
        ),
        out_shape=jax.ShapeDtypeStruct((n_pad, h), jnp.bfloat16),
    )(block_cat, block_valid, xs, W1, b1)

    th = h // 2
    u = pl.pallas_call(
        _k2_body,
        grid_spec=pltpu.PrefetchScalarGridSpec(
            num_scalar_prefetch=2,
            grid=(2, nb),
            in_specs=[
                pl.BlockSpec((B, h), lambda j, i, bc, bv: (i, 0)),
                pl.BlockSpec((1, h, th), lambda j, i, bc, bv: (bc[i], 0, j)),
                pl.BlockSpec((1, h, th), lambda j, i, bc, bv: (bc[i], 0, j)),
                pl.BlockSpec((1, 1, th), lambda j, i, bc, bv: (bc[i], 0, j)),
                pl.BlockSpec((1, 1, th), lambda j, i, bc, bv: (bc[i], 0, j)),
            ],
            out_specs=pl.BlockSpec((B, th), lambda j, i, bc, bv: (i, j)),
        ),
        out_shape=jax.ShapeDtypeStruct((n_pad, h), jnp.bfloat16),
    )(block_cat, block_valid, h1, Wm, Wg, bm, bg)

    h2 = pl.pallas_call(
        _k3_body,
        grid_spec=pltpu.PrefetchScalarGridSpec(
            num_scalar_prefetch=2,
            grid=(nb,),
            in_specs=[
                pl.BlockSpec((B, h), lambda i, bc, bv: (i, 0)),
                pl.BlockSpec((1, h, h), lambda i, bc, bv: (bc[i], 0, 0)),
                pl.BlockSpec((1, 1, h), lambda i, bc, bv: (bc[i], 0, 0)),
            ],
            out_specs=pl.BlockSpec((B, h), lambda i, bc, bv: (i, 0)),
        ),
        out_shape=jax.ShapeDtypeStruct((n_pad, h), jnp.bfloat16),
    )(block_cat, block_valid, u, Wo, bo)

    ys = pl.pallas_call(
        _k4_body,
        grid_spec=pltpu.PrefetchScalarGridSpec(
            num_scalar_prefetch=2,
            grid=(nb,),
            in_specs=[
                pl.BlockSpec((B, h), lambda i, bc, bv: (i, 0)),
                pl.BlockSpec((1, h, d), lambda i, bc, bv: (bc[i], 0, 0)),
                pl.BlockSpec((1, 1, d), lambda i, bc, bv: (bc[i], 0, 0)),
                pl.BlockSpec((B, d), lambda i, bc, bv: (i, 0)),
            ],
            out_specs=pl.BlockSpec((B, d), lambda i, bc, bv: (i, 0)),
        ),
        out_shape=jax.ShapeDtypeStruct((n_pad, d), jnp.float32),
    )(block_cat, block_valid, h2, W2, b2, xs)

    return _sc_gather_rows(ys, slot)


# speedup vs baseline: 1.2035x; 1.0381x over previous
import functools

import jax
import jax.numpy as jnp
from jax import lax
from jax.experimental import pallas as pl
from jax.experimental.pallas import tpu as pltpu
from jax.experimental.pallas import tpu_sc as plsc

B = 256


def _ln(v, eps=1e-5):
    mu = jnp.mean(v, axis=-1, keepdims=True)
    var = jnp.mean((v - mu) ** 2, axis=-1, keepdims=True)
    return (v - mu) * lax.rsqrt(var + eps)


def _sc_gather_rows(table, idx):
    rows_out = idx.shape[0]
    d = table.shape[1]
    info = plsc.get_sparse_core_info()
    nw = info.num_cores * info.num_subcores
    rpw = rows_out // nw
    assert rows_out % nw == 0 and rpw % 8 == 0
    max_chunk = (448 * 1024) // (d * table.dtype.itemsize)
    chunk = max(k for k in range(8, max_chunk + 1, 8) if rpw % k == 0)
    nch = rpw // chunk
    mesh = plsc.VectorSubcoreMesh(core_axis_name="c", subcore_axis_name="s")

    @functools.partial(
        pl.kernel,
        mesh=mesh,
        out_type=jax.ShapeDtypeStruct((rows_out, d), table.dtype),
        scratch_types=[
            pltpu.VMEM((chunk,), jnp.int32),
            pltpu.VMEM((chunk, d), table.dtype),
            pltpu.SemaphoreType.DMA,
        ],
    )
    def k(table_hbm, idx_hbm, out_hbm, idx_v, rows_v, sem):
        wid = lax.axis_index("s") * info.num_cores + lax.axis_index("c")
        base = wid * rpw

        def body(i, carry):
            off = base + i * chunk
            pltpu.sync_copy(idx_hbm.at[pl.ds(off, chunk)], idx_v)
            pltpu.async_copy(table_hbm.at[idx_v], rows_v, sem).wait()
            pltpu.sync_copy(rows_v, out_hbm.at[pl.ds(off, chunk)])
            return carry

        lax.fori_loop(0, nch, body, 0)

    return k(table, idx)


def _k1_body(bc_ref, bv_ref, xs_ref, w1_ref, b1_ref, o_ref):
    @pl.when(bv_ref[pl.program_id(0)] > 0)
    def _():
        xn = _ln(xs_ref[...])
        h = jnp.dot(xn, w1_ref[0], preferred_element_type=jnp.float32)
        o_ref[...] = jnp.maximum(h + b1_ref[0], 0.0).astype(jnp.bfloat16)


def _k2_body(bc_ref, bv_ref, h1_ref, wm_ref, wg_ref, bm_ref, bg_ref, o_ref):
    @pl.when(bv_ref[pl.program_id(1)] > 0)
    def _():
        h1 = h1_ref[...].astype(jnp.float32)
        main = jnp.dot(h1, wm_ref[0], preferred_element_type=jnp.float32)
        gate = jnp.dot(h1, wg_ref[0], preferred_element_type=jnp.float32)
        main = main + bm_ref[0]
        gate = gate + bg_ref[0]
        o_ref[...] = (main * jax.nn.sigmoid(gate)).astype(jnp.bfloat16)


def _k34_body(bc_ref, bv_ref, u_ref, wo_ref, bo_ref, w2_ref, b2_ref, xs_ref,
              o_ref):
    @pl.when(bv_ref[pl.program_id(0)] > 0)
    def _():
        g = _ln(u_ref[...].astype(jnp.float32))
        t = jnp.dot(g, wo_ref[0], preferred_element_type=jnp.float32)
        hn = _ln(t + bo_ref[0])
        o = jnp.dot(hn, w2_ref[0], preferred_element_type=jnp.float32)
        o = o + b2_ref[0] + 0.1 * xs_ref[...]
        o_ref[...] = _ln(o)


def kernel(x, cat_ids, W1, b1, Wm, bm, Wg, bg, Wo, bo, W2, b2):
    n, d = x.shape
    c, _, h = W1.shape
    b1, bm, bg, bo, b2 = (v[:, None, :] for v in (b1, bm, bg, bo, b2))
    n_pad = n + c * B
    nb = n_pad // B

    cat = cat_ids.astype(jnp.int32)
    onehot = (cat[:, None] == jnp.arange(c, dtype=jnp.int32)[None, :])
    ranks = jnp.cumsum(onehot.astype(jnp.int32), axis=0)
    counts = ranks[-1]
    rank = jnp.take_along_axis(ranks, cat[:, None], axis=1)[:, 0] - 1
    padded = ((counts + B - 1) // B) * B
    pad_start = jnp.concatenate(
        [jnp.zeros((1,), jnp.int32), jnp.cumsum(padded)[:-1].astype(jnp.int32)])
    slot = pad_start[cat] + rank
    src_idx = (jnp.arange(n_pad, dtype=jnp.int32) % n).at[slot].set(
        jnp.arange(n, dtype=jnp.int32))
    blocks_end = jnp.cumsum(padded // B).astype(jnp.int32)
    block_cat = jnp.minimum(
        jnp.searchsorted(blocks_end, jnp.arange(nb, dtype=jnp.int32),
                         side="right"),
        c - 1).astype(jnp.int32)
    block_valid = (jnp.arange(nb, dtype=jnp.int32)
                   < blocks_end[-1]).astype(jnp.int32)

    xs = _sc_gather_rows(x, src_idx)

    h1 = pl.pallas_call(
        _k1_body,
        grid_spec=pltpu.PrefetchScalarGridSpec(
            num_scalar_prefetch=2,
            grid=(nb,),
            in_specs=[
                pl.BlockSpec((B, d), lambda i, bc, bv: (i, 0)),
                pl.BlockSpec((1, d, h), lambda i, bc, bv: (bc[i], 0, 0)),
                pl.BlockSpec((1, 1, h), lambda i, bc, bv: (bc[i], 0, 0)),
            ],
            out_specs=pl.BlockSpec((B, h), lambda i, bc, bv: (i, 0)),
        ),
        out_shape=jax.ShapeDtypeStruct((n_pad, h), jnp.bfloat16),
    )(block_cat, block_valid, xs, W1, b1)

    th = h // 2
    u = pl.pallas_call(
        _k2_body,
        grid_spec=pltpu.PrefetchScalarGridSpec(
            num_scalar_prefetch=2,
            grid=(2, nb),
            in_specs=[
                pl.BlockSpec((B, h), lambda j, i, bc, bv: (i, 0)),
                pl.BlockSpec((1, h, th), lambda j, i, bc, bv: (bc[i], 0, j)),
                pl.BlockSpec((1, h, th), lambda j, i, bc, bv: (bc[i], 0, j)),
                pl.BlockSpec((1, 1, th), lambda j, i, bc, bv: (bc[i], 0, j)),
                pl.BlockSpec((1, 1, th), lambda j, i, bc, bv: (bc[i], 0, j)),
            ],
            out_specs=pl.BlockSpec((B, th), lambda j, i, bc, bv: (i, j)),
        ),
        out_shape=jax.ShapeDtypeStruct((n_pad, h), jnp.bfloat16),
    )(block_cat, block_valid, h1, Wm, Wg, bm, bg)

    ys = pl.pallas_call(
        _k34_body,
        grid_spec=pltpu.PrefetchScalarGridSpec(
            num_scalar_prefetch=2,
            grid=(nb,),
            in_specs=[
                pl.BlockSpec((B, h), lambda i, bc, bv: (i, 0)),
                pl.BlockSpec((1, h, h), lambda i, bc, bv: (bc[i], 0, 0)),
                pl.BlockSpec((1, 1, h), lambda i, bc, bv: (bc[i], 0, 0)),
                pl.BlockSpec((1, h, d), lambda i, bc, bv: (bc[i], 0, 0)),
                pl.BlockSpec((1, 1, d), lambda i, bc, bv: (bc[i], 0, 0)),
                pl.BlockSpec((B, d), lambda i, bc, bv: (i, 0)),
            ],
            out_specs=pl.BlockSpec((B, d), lambda i, bc, bv: (i, 0)),
        ),
        out_shape=jax.ShapeDtypeStruct((n_pad, d), jnp.float32),
    )(block_cat, block_valid, u, Wo, bo, W2, b2, xs)

    return _sc_gather_rows(ys, slot)


# speedup vs baseline: 1.2620x; 1.0486x over previous
import functools

import jax
import jax.numpy as jnp
from jax import lax
from jax.experimental import pallas as pl
from jax.experimental.pallas import tpu as pltpu
from jax.experimental.pallas import tpu_sc as plsc

B = 256


def _ln(v, eps=1e-5):
    mu = jnp.mean(v, axis=-1, keepdims=True)
    var = jnp.mean((v - mu) ** 2, axis=-1, keepdims=True)
    return (v - mu) * lax.rsqrt(var + eps)


def _sc_gather_rows(table, idx):
    rows_out = idx.shape[0]
    d = table.shape[1]
    info = plsc.get_sparse_core_info()
    nw = info.num_cores * info.num_subcores
    rpw = rows_out // nw
    assert rows_out % nw == 0 and rpw % 8 == 0
    max_chunk = (448 * 1024) // (d * table.dtype.itemsize)
    chunk = max(k for k in range(8, max_chunk + 1, 8) if rpw % k == 0)
    nch = rpw // chunk
    mesh = plsc.VectorSubcoreMesh(core_axis_name="c", subcore_axis_name="s")

    @functools.partial(
        pl.kernel,
        mesh=mesh,
        out_type=jax.ShapeDtypeStruct((rows_out, d), table.dtype),
        scratch_types=[
            pltpu.VMEM((chunk,), jnp.int32),
            pltpu.VMEM((chunk, d), table.dtype),
            pltpu.SemaphoreType.DMA,
        ],
    )
    def k(table_hbm, idx_hbm, out_hbm, idx_v, rows_v, sem):
        wid = lax.axis_index("s") * info.num_cores + lax.axis_index("c")
        base = wid * rpw

        def body(i, carry):
            off = base + i * chunk
            pltpu.sync_copy(idx_hbm.at[pl.ds(off, chunk)], idx_v)
            pltpu.async_copy(table_hbm.at[idx_v], rows_v, sem).wait()
            pltpu.sync_copy(rows_v, out_hbm.at[pl.ds(off, chunk)])
            return carry

        lax.fori_loop(0, nch, body, 0)

    return k(table, idx)


def _prefetch_weights(i, nsteps, r, first, runcat_ref, srcs, wbufs, sem):
    @pl.when(i == 0)
    def _():
        for s, wb in zip(srcs, wbufs):
            pltpu.make_async_copy(s(runcat_ref[0]), wb.at[0], sem.at[0]).start()

    @pl.when(first)
    def _():
        nxt = r + 1
        for s, wb in zip(srcs, wbufs):
            pltpu.make_async_copy(
                s(runcat_ref[nxt]), wb.at[nxt % 2], sem.at[nxt % 2]).start()
        for s, wb in zip(srcs, wbufs):
            pltpu.make_async_copy(
                s(runcat_ref[r]), wb.at[r % 2], sem.at[r % 2]).wait()

    @pl.when(i == nsteps - 1)
    def _():
        nxt = r + 1
        for s, wb in zip(srcs, wbufs):
            pltpu.make_async_copy(
                s(runcat_ref[nxt]), wb.at[nxt % 2], sem.at[nxt % 2]).wait()


def _k1_body(run_ref, runcat_ref, bv_ref, xs_ref, w1_any, b1_ref, o_ref,
             wbuf, sem):
    i = pl.program_id(0)
    r = run_ref[i]
    first = jnp.logical_or(i == 0, run_ref[jnp.maximum(i - 1, 0)] != r)
    _prefetch_weights(i, pl.num_programs(0), r, first, runcat_ref,
                      [lambda c: w1_any.at[c]], [wbuf], sem)

    @pl.when(bv_ref[i] > 0)
    def _():
        xn = _ln(xs_ref[...])
        h = jnp.dot(xn, wbuf[r % 2], preferred_element_type=jnp.float32)
        o_ref[...] = jnp.maximum(h + b1_ref[0], 0.0).astype(jnp.bfloat16)


def _k2_body(bc_ref, bv_ref, h1_ref, wm_ref, wg_ref, bm_ref, bg_ref, o_ref):
    @pl.when(bv_ref[pl.program_id(1)] > 0)
    def _():
        h1 = h1_ref[...].astype(jnp.float32)
        main = jnp.dot(h1, wm_ref[0], preferred_element_type=jnp.float32)
        gate = jnp.dot(h1, wg_ref[0], preferred_element_type=jnp.float32)
        main = main + bm_ref[0]
        gate = gate + bg_ref[0]
        o_ref[...] = (main * jax.nn.sigmoid(gate)).astype(jnp.bfloat16)


def _k34_body(run_ref, runcat_ref, bv_ref, u_ref, wo_any, bo_ref, w2_any,
              b2_ref, xs_ref, o_ref, wobuf, w2buf, sem):
    i = pl.program_id(0)
    r = run_ref[i]
    first = jnp.logical_or(i == 0, run_ref[jnp.maximum(i - 1, 0)] != r)
    _prefetch_weights(i, pl.num_programs(0), r, first, runcat_ref,
                      [lambda c: wo_any.at[c], lambda c: w2_any.at[c]],
                      [wobuf, w2buf], sem)

    @pl.when(bv_ref[i] > 0)
    def _():
        g = _ln(u_ref[...].astype(jnp.float32))
        t = jnp.dot(g, wobuf[r % 2], preferred_element_type=jnp.float32)
        hn = _ln(t + bo_ref[0])
        o = jnp.dot(hn, w2buf[r % 2], preferred_element_type=jnp.float32)
        o = o + b2_ref[0] + 0.1 * xs_ref[...]
        o_ref[...] = _ln(o)


def kernel(x, cat_ids, W1, b1, Wm, bm, Wg, bg, Wo, bo, W2, b2):
    n, d = x.shape
    c, _, h = W1.shape
    b1, bm, bg, bo, b2 = (v[:, None, :] for v in (b1, bm, bg, bo, b2))
    n_pad = n + c * B
    nb = n_pad // B

    cat = cat_ids.astype(jnp.int32)
    onehot = (cat[:, None] == jnp.arange(c, dtype=jnp.int32)[None, :])
    ranks = jnp.cumsum(onehot.astype(jnp.int32), axis=0)
    counts = ranks[-1]
    rank = jnp.take_along_axis(ranks, cat[:, None], axis=1)[:, 0] - 1
    padded = ((counts + B - 1) // B) * B
    pad_start = jnp.concatenate(
        [jnp.zeros((1,), jnp.int32), jnp.cumsum(padded)[:-1].astype(jnp.int32)])
    slot = pad_start[cat] + rank
    src_idx = (jnp.arange(n_pad, dtype=jnp.int32) % n).at[slot].set(
        jnp.arange(n, dtype=jnp.int32))
    blocks_end = jnp.cumsum(padded // B).astype(jnp.int32)
    block_cat = jnp.minimum(
        jnp.searchsorted(blocks_end, jnp.arange(nb, dtype=jnp.int32),
                         side="right"),
        c - 1).astype(jnp.int32)
    block_valid = (jnp.arange(nb, dtype=jnp.int32)
                   < blocks_end[-1]).astype(jnp.int32)
    chg = jnp.concatenate([jnp.ones((1,), jnp.bool_),
                           block_cat[1:] != block_cat[:-1]])
    run_id = jnp.cumsum(chg.astype(jnp.int32)) - 1
    run_cat = jnp.zeros((nb + 1,), jnp.int32).at[run_id].set(block_cat)

    xs = _sc_gather_rows(x, src_idx)

    h1 = pl.pallas_call(
        _k1_body,
        grid_spec=pltpu.PrefetchScalarGridSpec(
            num_scalar_prefetch=3,
            grid=(nb,),
            in_specs=[
                pl.BlockSpec((B, d), lambda i, ri, rc, bv: (i, 0)),
                pl.BlockSpec(memory_space=pltpu.MemorySpace.HBM),
                pl.BlockSpec((1, 1, h), lambda i, ri, rc, bv: (rc[ri[i]], 0, 0)),
            ],
            out_specs=pl.BlockSpec((B, h), lambda i, ri, rc, bv: (i, 0)),
            scratch_shapes=[pltpu.VMEM((2, d, h), jnp.float32),
                            pltpu.SemaphoreType.DMA((2,))],
        ),
        out_shape=jax.ShapeDtypeStruct((n_pad, h), jnp.bfloat16),
    )(run_id, run_cat, block_valid, xs, W1, b1)

    th = h // 2
    u = pl.pallas_call(
        _k2_body,
        grid_spec=pltpu.PrefetchScalarGridSpec(
            num_scalar_prefetch=2,
            grid=(2, nb),
            in_specs=[
                pl.BlockSpec((B, h), lambda j, i, bc, bv: (i, 0)),
                pl.BlockSpec((1, h, th), lambda j, i, bc, bv: (bc[i], 0, j)),
                pl.BlockSpec((1, h, th), lambda j, i, bc, bv: (bc[i], 0, j)),
                pl.BlockSpec((1, 1, th), lambda j, i, bc, bv: (bc[i], 0, j)),
                pl.BlockSpec((1, 1, th), lambda j, i, bc, bv: (bc[i], 0, j)),
            ],
            out_specs=pl.BlockSpec((B, th), lambda j, i, bc, bv: (i, j)),
        ),
        out_shape=jax.ShapeDtypeStruct((n_pad, h), jnp.bfloat16),
    )(block_cat, block_valid, h1, Wm, Wg, bm, bg)

    ys = pl.pallas_call(
        _k34_body,
        grid_spec=pltpu.PrefetchScalarGridSpec(
            num_scalar_prefetch=3,
            grid=(nb,),
            in_specs=[
                pl.BlockSpec((B, h), lambda i, ri, rc, bv: (i, 0)),
                pl.BlockSpec(memory_space=pltpu.MemorySpace.HBM),
                pl.BlockSpec((1, 1, h), lambda i, ri, rc, bv: (rc[ri[i]], 0, 0)),
                pl.BlockSpec(memory_space=pltpu.MemorySpace.HBM),
                pl.BlockSpec((1, 1, d), lambda i, ri, rc, bv: (rc[ri[i]], 0, 0)),
                pl.BlockSpec((B, d), lambda i, ri, rc, bv: (i, 0)),
            ],
            out_specs=pl.BlockSpec((B, d), lambda i, ri, rc, bv: (i, 0)),
            scratch_shapes=[pltpu.VMEM((2, h, h), jnp.float32),
                            pltpu.VMEM((2, h, d), jnp.float32),
                            pltpu.SemaphoreType.DMA((2,))],
        ),
        out_shape=jax.ShapeDtypeStruct((n_pad, d), jnp.float32),
    )(run_id, run_cat, block_valid, u, Wo, bo, W2, b2, xs)

    return _sc_gather_rows(ys, slot)
